# baseline (speedup 1.0000x reference)
import functools
import math

import jax
import jax.numpy as jnp
from jax import lax
from jax.experimental import pallas as pl
from jax.experimental.pallas import tpu as pltpu
from jax.experimental.pallas import tpu_sc as plsc

_N = 8192
_LANES = 16
_VREGS = _N // _LANES

_LOGN = math.log(float(_N))
_T1 = float(0.3 * _LOGN)
_T2 = float(0.6 * _LOGN)
_T3 = float(0.7 * _LOGN)
_C21 = float(math.exp(_T2 - _T1))
_C31 = float(math.exp(_T3 - _T1))


def _xlane(v, op):
    for d in (1, 2, 4, 8):
        idx = lax.iota(jnp.int32, _LANES) ^ d
        v = op(v, v[idx])
    return v


_mesh = plsc.VectorSubcoreMesh(
    core_axis_name="c", subcore_axis_name="s", num_cores=1)


@functools.partial(
    pl.kernel,
    mesh=_mesh,
    out_type=jax.ShapeDtypeStruct((_LANES,), jnp.int32),
    scratch_types=[
        pltpu.VMEM((_N,), jnp.float32),
        pltpu.VMEM((_LANES,), jnp.int32),
    ],
)
def _entropy_topk(scores_hbm, out_hbm, chunk_v, out_v):
    sid = lax.axis_index("s")

    @pl.when(sid == 0)
    def _():
        pltpu.sync_copy(scores_hbm, chunk_v)
        acc = [jnp.zeros((_LANES,), jnp.float32) for _ in range(8)]
        for i in range(0, _VREGS, 4):
            for j in range(4):
                x = chunk_v[pl.ds((i + j) * _LANES, _LANES)]
                e = jnp.exp(x)
                acc[2 * j] = acc[2 * j] + e
                acc[2 * j + 1] = acc[2 * j + 1] + e * x
        S = (acc[0] + acc[2]) + (acc[4] + acc[6])
        A = (acc[1] + acc[3]) + (acc[5] + acc[7])
        sv = _xlane(S, jnp.add)
        rv = _xlane(A, jnp.add) / sv
        e1 = jnp.exp(rv + _T1)
        c1 = sv < e1
        c2 = sv < e1 * _C21
        c3 = sv < e1 * _C31
        k2 = jnp.full((_LANES,), 2, jnp.int32)
        k4 = jnp.full((_LANES,), 4, jnp.int32)
        k8 = jnp.full((_LANES,), 8, jnp.int32)
        k32 = jnp.full((_LANES,), 32, jnp.int32)
        kv = jnp.where(c1, k2, jnp.where(c2, k4, jnp.where(c3, k8, k32)))
        out_v[...] = kv
        pltpu.sync_copy(out_v, out_hbm)


def kernel(scores):
    return _entropy_topk(scores)[0]


# speedup vs baseline: 1.1270x; 1.1270x over previous
import functools
import math

import jax
import jax.numpy as jnp
from jax import lax
from jax.experimental import pallas as pl
from jax.experimental.pallas import tpu as pltpu
from jax.experimental.pallas import tpu_sc as plsc

_N = 8192
_LANES = 16
_NSUB = 16
_CHUNK = _N // _NSUB
_VPT = _CHUNK // _LANES

_LOGN = math.log(float(_N))
_T1 = float(0.3 * _LOGN)
_T2 = float(0.6 * _LOGN)
_T3 = float(0.7 * _LOGN)
_C21 = float(math.exp(_T2 - _T1))
_C31 = float(math.exp(_T3 - _T1))


def _xlane(v, op):
    for d in (1, 2, 4, 8):
        idx = lax.iota(jnp.int32, _LANES) ^ d
        v = op(v, v[idx])
    return v


_mesh = plsc.VectorSubcoreMesh(
    core_axis_name="c", subcore_axis_name="s", num_cores=1)


@functools.partial(
    pl.kernel,
    mesh=_mesh,
    out_type=jax.ShapeDtypeStruct((_LANES,), jnp.int32),
    scratch_types=[
        pltpu.VMEM((_CHUNK,), jnp.float32),
        pltpu.VMEM((2, _LANES), jnp.float32),
        pltpu.VMEM_SHARED((_NSUB, 2, _LANES), jnp.float32),
        pltpu.VMEM((_NSUB, 2, _LANES), jnp.float32),
        pltpu.VMEM((_LANES,), jnp.int32),
    ],
)
def _entropy_topk(scores_hbm, out_hbm, chunk_v, stage_v, shared, all_v, out_v):
    sid = lax.axis_index("s")

    pltpu.sync_copy(scores_hbm.at[pl.ds(sid * _CHUNK, _CHUNK)], chunk_v)

    s0 = jnp.zeros((_LANES,), jnp.float32)
    a0 = jnp.zeros((_LANES,), jnp.float32)
    s1 = jnp.zeros((_LANES,), jnp.float32)
    a1 = jnp.zeros((_LANES,), jnp.float32)
    for i in range(0, _VPT, 2):
        x0 = chunk_v[pl.ds(i * _LANES, _LANES)]
        x1 = chunk_v[pl.ds((i + 1) * _LANES, _LANES)]
        e0 = jnp.exp(x0)
        e1 = jnp.exp(x1)
        s0 = s0 + e0
        a0 = a0 + e0 * x0
        s1 = s1 + e1
        a1 = a1 + e1 * x1

    stage_v[0, :] = s0 + s1
    stage_v[1, :] = a0 + a1
    pltpu.sync_copy(stage_v, shared.at[sid])
    plsc.subcore_barrier()

    @pl.when(sid == 0)
    def _():
        pltpu.sync_copy(shared, all_v)
        S = all_v[0, 0, :]
        A = all_v[0, 1, :]
        for w in range(1, _NSUB):
            S = S + all_v[w, 0, :]
            A = A + all_v[w, 1, :]
        sv = _xlane(S, jnp.add)
        rv = _xlane(A, jnp.add) / sv
        e1 = jnp.exp(rv + _T1)
        c1 = sv < e1
        c2 = sv < e1 * _C21
        c3 = sv < e1 * _C31
        k2 = jnp.full((_LANES,), 2, jnp.int32)
        k4 = jnp.full((_LANES,), 4, jnp.int32)
        k8 = jnp.full((_LANES,), 8, jnp.int32)
        k32 = jnp.full((_LANES,), 32, jnp.int32)
        kv = jnp.where(c1, k2, jnp.where(c2, k4, jnp.where(c3, k8, k32)))
        out_v[...] = kv
        pltpu.sync_copy(out_v, out_hbm)


def kernel(scores):
    return _entropy_topk(scores)[0]
